# Initial kernel scaffold; baseline (speedup 1.0000x reference)
import jax
import jax.numpy as jnp
from jax.experimental import pallas as pl


def kernel(x, hist, logp_ref):
    raise NotImplementedError("write your pallas kernel here")


# speedup vs baseline: 146.2205x; 146.2205x over previous
import functools

import jax
import jax.numpy as jnp
from jax import lax
from jax.experimental import pallas as pl
from jax.experimental.pallas import tpu as pltpu
from jax.experimental.pallas import tpu_sc as plsc

NC, NS, LANES = 2, 16, 16
NW = NC * NS
C = 5
LROW = 4194304
NTOT = C * LROW
PER_TILE = LROW // NW
CHUNK = 8192
NCHUNK = PER_TILE // CHUNK
RANK = 4152360

_MESH = plsc.VectorSubcoreMesh(
    core_axis_name="c", subcore_axis_name="s", num_cores=NC, num_subcores=NS)


def _stream_rows(x_hbm, buf, sems, wid, body, row_prologue, row_epilogue):
  for r in range(C):
    row_prologue(r)
    base = r * LROW + wid * PER_TILE
    pltpu.async_copy(x_hbm.at[pl.ds(base, CHUNK)], buf.at[0], sems[0])
    pltpu.async_copy(x_hbm.at[pl.ds(base + CHUNK, CHUNK)], buf.at[1], sems[1])

    @pl.loop(0, NCHUNK, step=2)
    def _(g2):
      for b in range(2):
        g = g2 + b
        pltpu.make_async_copy(
            x_hbm.at[pl.ds(0, CHUNK)], buf.at[b], sems[b]).wait()

        @pl.loop(0, CHUNK // LANES, unroll=8)
        def _(i):
          body(r, b, g, i)

        nxt = g + 2

        @pl.when(nxt < NCHUNK)
        def _():
          pltpu.async_copy(
              x_hbm.at[pl.ds(base + nxt * CHUNK, CHUNK)], buf.at[b], sems[b])

    row_epilogue(r)


def _make_refine(shift, nbits, masked):
  nbins = 1 << nbits
  hsz = LANES * nbins

  scratch = [
      pltpu.VMEM((2, CHUNK), jnp.float32),
      pltpu.VMEM((hsz,), jnp.int32),
      pltpu.VMEM((nbins,), jnp.int32),
  ]
  if masked:
    scratch.append(pltpu.VMEM((C * LANES,), jnp.int32))
  scratch += [pltpu.SemaphoreType.DMA, pltpu.SemaphoreType.DMA]

  @functools.partial(
      pl.kernel,
      out_type=jax.ShapeDtypeStruct((NW * C * nbins,), jnp.int32),
      mesh=_MESH,
      compiler_params=pltpu.CompilerParams(needs_layout_passes=False),
      scratch_types=scratch,
  )
  def kfn(*args):
    if masked:
      x_hbm, prevp_hbm, out_hbm, buf, hist, merged, prevp_v, sem0, sem1 = args
    else:
      x_hbm, out_hbm, buf, hist, merged, sem0, sem1 = args
      prevp_v = None
    wid = lax.axis_index("c") * NS + lax.axis_index("s")
    if masked:
      pltpu.sync_copy(prevp_hbm, prevp_v)
    lane_base = lax.iota(jnp.int32, LANES) * nbins
    ones = jnp.ones((LANES,), jnp.int32)
    zer = jnp.zeros((LANES,), jnp.int32)
    row_state = {}

    def pro(r):
      @pl.loop(0, hsz // LANES, unroll=8)
      def _(j):
        hist[pl.ds(j * LANES, LANES)] = zer
      if masked:
        row_state["prev"] = prevp_v[pl.ds(r * LANES, LANES)]

    def body(r, b, g, i):
      v = buf[b, pl.ds(i * LANES, LANES)]
      p = lax.bitcast_convert_type(v, jnp.int32) & 0x7FFFFFFF
      digit = ((p >> shift) & (nbins - 1)) + lane_base
      if masked:
        m = (p >> (shift + nbits)) == row_state["prev"]
        plsc.addupdate_scatter(hist, [digit], ones, mask=m)
      else:
        plsc.addupdate_scatter(hist, [digit], ones)

    def epi(r):
      @pl.loop(0, nbins // LANES)
      def _(jb):
        o = jb * LANES
        acc = hist[pl.ds(o, LANES)]
        for l in range(1, LANES):
          acc = acc + hist[pl.ds(l * nbins + o, LANES)]
        merged[pl.ds(o, LANES)] = acc
      pltpu.sync_copy(merged, out_hbm.at[pl.ds((wid * C + r) * nbins, nbins)])

    _stream_rows(x_hbm, buf, (sem0, sem1), wid, body, pro, epi)

  return kfn


def _bin_index(v, mv):
  t = jnp.abs(v) / mv
  n = t * jnp.float32(8.0)
  cc = jnp.minimum(n, jnp.float32(8.0))
  d = (cc * jnp.float32(0.125)) * jnp.float32(255.0)
  ix = d.astype(jnp.int32)
  return jnp.minimum(jnp.maximum(ix, 0), 255)


def _make_hist256():
  nbins = 256
  hsz = LANES * nbins

  @functools.partial(
      pl.kernel,
      out_type=jax.ShapeDtypeStruct((NW * C * nbins,), jnp.int32),
      mesh=_MESH,
      compiler_params=pltpu.CompilerParams(needs_layout_passes=False),
      scratch_types=[
          pltpu.VMEM((2, CHUNK), jnp.float32),
          pltpu.VMEM((hsz,), jnp.int32),
          pltpu.VMEM((nbins,), jnp.int32),
          pltpu.VMEM((C * LANES,), jnp.float32),
          pltpu.SemaphoreType.DMA,
          pltpu.SemaphoreType.DMA,
      ],
  )
  def kfn(x_hbm, maxv_hbm, out_hbm, buf, hist, merged, maxv_v, sem0, sem1):
    wid = lax.axis_index("c") * NS + lax.axis_index("s")
    pltpu.sync_copy(maxv_hbm, maxv_v)
    lane_base = lax.iota(jnp.int32, LANES) * nbins
    ones = jnp.ones((LANES,), jnp.int32)
    zer = jnp.zeros((LANES,), jnp.int32)
    row_state = {}

    def pro(r):
      @pl.loop(0, hsz // LANES, unroll=8)
      def _(j):
        hist[pl.ds(j * LANES, LANES)] = zer
      row_state["mv"] = maxv_v[pl.ds(r * LANES, LANES)]

    def body(r, b, g, i):
      v = buf[b, pl.ds(i * LANES, LANES)]
      ix = _bin_index(v, row_state["mv"]) + lane_base
      plsc.addupdate_scatter(hist, [ix], ones)

    def epi(r):
      @pl.loop(0, nbins // LANES)
      def _(jb):
        o = jb * LANES
        acc = hist[pl.ds(o, LANES)]
        for l in range(1, LANES):
          acc = acc + hist[pl.ds(l * nbins + o, LANES)]
        merged[pl.ds(o, LANES)] = acc
      pltpu.sync_copy(merged, out_hbm.at[pl.ds((wid * C + r) * nbins, nbins)])

    _stream_rows(x_hbm, buf, (sem0, sem1), wid, body, pro, epi)

  return kfn


def _make_apply():
  @functools.partial(
      pl.kernel,
      out_type=jax.ShapeDtypeStruct((NTOT,), jnp.float32),
      mesh=_MESH,
      compiler_params=pltpu.CompilerParams(needs_layout_passes=False),
      scratch_types=[
          pltpu.VMEM((2, CHUNK), jnp.float32),
          pltpu.VMEM((2, CHUNK), jnp.float32),
          pltpu.VMEM((C * 256,), jnp.float32),
          pltpu.VMEM((C * LANES,), jnp.float32),
          pltpu.SemaphoreType.DMA,
          pltpu.SemaphoreType.DMA,
          pltpu.SemaphoreType.DMA,
          pltpu.SemaphoreType.DMA,
      ],
  )
  def kfn(x_hbm, mtab_hbm, maxv_hbm, out_hbm, buf, obuf, mtab_v, maxv_v,
          si0, si1, so0, so1):
    wid = lax.axis_index("c") * NS + lax.axis_index("s")
    pltpu.sync_copy(mtab_hbm, mtab_v)
    pltpu.sync_copy(maxv_hbm, maxv_v)
    osems = (so0, so1)
    row_state = {}

    def pro(r):
      row_state["mv"] = maxv_v[pl.ds(r * LANES, LANES)]

    def body(r, b, g, i):
      v = buf[b, pl.ds(i * LANES, LANES)]
      ix = _bin_index(v, row_state["mv"]) + (r * 256)
      gt = plsc.load_gather(mtab_v, [ix])
      obuf[b, pl.ds(i * LANES, LANES)] = v * gt

    for r in range(C):
      pro(r)
      base = r * LROW + wid * PER_TILE
      pltpu.async_copy(x_hbm.at[pl.ds(base, CHUNK)], buf.at[0], si0)
      pltpu.async_copy(x_hbm.at[pl.ds(base + CHUNK, CHUNK)], buf.at[1], si1)
      isems = (si0, si1)

      @pl.loop(0, NCHUNK, step=2)
      def _(g2):
        for b in range(2):
          g = g2 + b
          pltpu.make_async_copy(
              x_hbm.at[pl.ds(0, CHUNK)], buf.at[b], isems[b]).wait()

          @pl.when(g >= 2)
          def _():
            pltpu.make_async_copy(
                obuf.at[b], out_hbm.at[pl.ds(0, CHUNK)], osems[b]).wait()

          @pl.loop(0, CHUNK // LANES, unroll=8)
          def _(i):
            body(r, b, g, i)

          pltpu.async_copy(
              obuf.at[b], out_hbm.at[pl.ds(base + g * CHUNK, CHUNK)], osems[b])
          nxt = g + 2

          @pl.when(nxt < NCHUNK)
          def _():
            pltpu.async_copy(
                x_hbm.at[pl.ds(base + nxt * CHUNK, CHUNK)], buf.at[b],
                isems[b])

      for b in range(2):
        pltpu.make_async_copy(
            obuf.at[b], out_hbm.at[pl.ds(0, CHUNK)], osems[b]).wait()

  return kfn


_L1 = _make_refine(20, 11, masked=False)
_L2 = _make_refine(9, 11, masked=True)
_L3 = _make_refine(0, 9, masked=True)
_H256 = _make_hist256()
_APPLY = _make_apply()


def _pick(cnt, rank):
  cum = jnp.cumsum(cnt, axis=1)
  b = jnp.argmax(cum >= (rank[:, None] + 1), axis=1).astype(jnp.int32)
  cumprev = jnp.take_along_axis(cum - cnt, b[:, None], axis=1)[:, 0]
  return b, rank - cumprev


def kernel(x, hist, logp_ref):
  xf = jnp.reshape(x, (-1,))

  cnt1 = jnp.sum(jnp.reshape(_L1(xf), (NW, C, 2048)), axis=0)
  b1, r1 = _pick(cnt1, jnp.full((C,), RANK, jnp.int32))

  prev2 = jnp.reshape(jnp.broadcast_to(b1[:, None], (C, LANES)), (-1,))
  cnt2 = jnp.sum(jnp.reshape(_L2(xf, prev2), (NW, C, 2048)), axis=0)
  b2, r2 = _pick(cnt2, r1)

  prev3 = jnp.reshape(
      jnp.broadcast_to(((b1 << 11) | b2)[:, None], (C, LANES)), (-1,))
  cnt3 = jnp.sum(jnp.reshape(_L3(xf, prev3), (NW, C, 512)), axis=0)
  b3, _ = _pick(cnt3, r2)

  bits = (b1 << 20) | (b2 << 9) | b3
  maxv = jnp.maximum(
      lax.bitcast_convert_type(bits.astype(jnp.int32), jnp.float32),
      jnp.float32(1e-8))
  maxv_b = jnp.reshape(jnp.broadcast_to(maxv[:, None], (C, LANES)), (-1,))

  counts = jnp.sum(
      jnp.reshape(_H256(xf, maxv_b), (NW, C, 256)), axis=0).astype(jnp.float32)

  hist2 = (1.0 - 0.02) * hist + 0.02 * counts
  sm = hist2 + 1e-8
  logp_obs = jnp.log(sm / jnp.sum(sm, axis=-1, keepdims=True))
  mask_tab = jax.nn.sigmoid(-1.0 * ((logp_ref - logp_obs) - (-2.0)))

  out = _APPLY(xf, jnp.reshape(mask_tab, (-1,)), maxv_b)
  return jnp.reshape(out, x.shape)
